# baseline (speedup 1.0000x reference)
import functools

import jax
import jax.numpy as jnp
from jax import lax
from jax.experimental import pallas as pl
from jax.experimental.pallas import tpu as pltpu

BATCH = 1024
N_ACT = 100000

ROWS = 128
COLT = 4096
RSUB = 8
CSUB = 2048
CBLOCKS = (N_ACT + COLT - 1) // COLT
SB = (CBLOCKS - 1).bit_length()
KEYMASK = ((2**23 - 1) << SB) & 0x7FFFFFFF

_KS0 = 0
_KS1 = 42
_KS2 = 42 ^ 0x1BD11BDA
_ROT_A = (13, 15, 26, 6)
_ROT_B = (17, 29, 16, 24)
_INJ = ((_KS1, _KS2 + 1), (_KS2, _KS0 + 2), (_KS0, _KS1 + 3),
        (_KS1, _KS2 + 4), (_KS2, _KS0 + 5))


def _rotl(x, d):
    return lax.shift_left(x, jnp.int32(d)) | lax.shift_right_logical(
        x, jnp.int32(32 - d))


def _threefry_bits(x1):
    x0 = x1
    x1 = _rotl(x1, _ROT_A[0]) ^ x0
    first = True
    for g in range(5):
        rots = _ROT_A if g % 2 == 0 else _ROT_B
        for r in rots:
            if first:
                first = False
                continue
            x0 = x0 + x1
            x1 = _rotl(x1, r) ^ x0
        a, b = _INJ[g]
        x0 = x0 + jnp.int32(a)
        x1 = x1 + jnp.int32(b)
    return x0 ^ x1


def _body(mask_ref, act_ref, lp_ref, key_acc, cnt_acc):
    r = pl.program_id(0)
    c = pl.program_id(1)

    @pl.when(c == 0)
    def _init():
        key_acc[...] = jnp.full((ROWS, COLT), -1, jnp.int32)
        cnt_acc[...] = jnp.zeros((ROWS, COLT), jnp.int32)

    lane = lax.broadcasted_iota(jnp.int32, (RSUB, CSUB), 1)
    iota0 = lax.broadcasted_iota(jnp.int32, (RSUB, CSUB), 0)
    revstrip = jnp.int32(CBLOCKS - 1) - c
    base2d = iota0 * jnp.int32(N_ACT) + lane + jnp.int32(_KS1)
    scal0 = r * jnp.int32(ROWS * N_ACT) + c * jnp.int32(COLT)
    nchunk = (ROWS // RSUB) * (COLT // CSUB)

    def make_chunk(guarded):
        def chunk(k, _):
            ri = pl.multiple_of((k // (COLT // CSUB)) * RSUB, RSUB)
            ci = pl.multiple_of((k % (COLT // CSUB)) * CSUB, 256)
            m = mask_ref[pl.ds(ri, RSUB), pl.ds(ci, CSUB)]
            if guarded:
                valid = m & (lane < (jnp.int32(N_ACT) - c * jnp.int32(COLT)
                                     - ci))
            else:
                valid = m
            bits = _threefry_bits(base2d + (scal0 + ri * jnp.int32(N_ACT)
                                            + ci))
            key = (lax.shift_right_logical(bits, jnp.int32(9 - SB))
                   & jnp.int32(KEYMASK)) | revstrip
            v = jnp.where(valid, key, jnp.int32(-1))
            ka = key_acc[pl.ds(ri, RSUB), pl.ds(ci, CSUB)]
            key_acc[pl.ds(ri, RSUB), pl.ds(ci, CSUB)] = jnp.maximum(ka, v)
            ca = cnt_acc[pl.ds(ri, RSUB), pl.ds(ci, CSUB)]
            cnt_acc[pl.ds(ri, RSUB), pl.ds(ci, CSUB)] = \
                ca + valid.astype(jnp.int32)
            return 0
        return chunk

    @pl.when(c < CBLOCKS - 1)
    def _main():
        lax.fori_loop(0, nchunk, make_chunk(False), 0, unroll=4)

    @pl.when(c == CBLOCKS - 1)
    def _tail():
        lax.fori_loop(0, nchunk, make_chunk(True), 0, unroll=4)

    @pl.when(c == CBLOCKS - 1)
    def _fin():
        lane_f = lax.broadcasted_iota(jnp.int32, (RSUB, COLT), 1)
        for ri in range(ROWS // RSUB):
            keys = key_acc[pl.ds(ri * RSUB, RSUB), :]
            bb = lax.shift_right_arithmetic(keys, jnp.int32(SB))
            strip = jnp.int32(CBLOCKS - 1) - (keys & jnp.int32(2**SB - 1))
            gcol = strip * jnp.int32(COLT) + lane_f
            mx = jnp.max(bb, axis=1, keepdims=True)
            act_ref[pl.ds(ri * RSUB, RSUB), :] = jnp.min(
                jnp.where(bb == mx, gcol, jnp.int32(2**30)),
                axis=1, keepdims=True)
            cnt = jnp.sum(cnt_acc[pl.ds(ri * RSUB, RSUB), :],
                          axis=1, keepdims=True)
            lp_ref[pl.ds(ri * RSUB, RSUB), :] = -jnp.log(
                cnt.astype(jnp.float32))


from jax.experimental.pallas import tpu_sc as plsc

TC_ROWS = 768
SC_ROWS = BATCH - TC_ROWS
NSUB = 32
KROW = SC_ROWS // NSUB
N_W = N_ACT // 4
ROW_W = 25088
PAD_W = ROW_W - N_W


def _sc_body(mask_hbm, out_hbm, buf0, buf1, stage, sem0, sem1):
    wid = lax.axis_index("s") * 2 + lax.axis_index("c")
    row0 = TC_ROWS + wid * KROW
    sems = [sem0, sem1]
    iota = lax.iota(jnp.int32, 16)
    iota4 = iota * jnp.int32(4)
    neg1 = jnp.full((16,), -1, jnp.int32)

    bufs = [buf0, buf1]

    def start_row(r):
        cur = r % 2
        row = row0 + r
        h = pltpu.async_copy(mask_hbm.at[pl.ds(row * ROW_W, ROW_W)],
                             bufs[cur], sems[cur])
        return (h,)

    pending = start_row(0)
    for r in range(KROW):
        cur = r % 2
        row = row0 + r
        for h in pending:
            h.wait()
        if r + 1 < KROW:
            pending = start_row(r + 1)
        rowc = row * jnp.int32(N_ACT) + jnp.int32(_KS1)

        def group(words, base, k4, carry):
            bbits, bcol, cnt = carry
            mbit = lax.shift_right_logical(
                words, jnp.int32(8 * k4)) & jnp.int32(1)
            col = base + jnp.int32(k4) + iota4
            ok = mbit == jnp.int32(1)
            cnt = cnt + mbit
            v = lax.shift_right_logical(_threefry_bits(rowc + col),
                                        jnp.int32(9))
            v = jnp.where(ok, v, neg1)
            upd = v > bbits
            bbits = jnp.where(upd, v, bbits)
            bcol = jnp.where(upd, col, bcol)
            return (bbits, bcol, cnt)

        buf = bufs[cur]

        def step(k, carry):
            base = k * jnp.int32(64)
            words = buf[pl.ds(pl.multiple_of(k * 16, 16), 16)]
            for k4 in range(4):
                carry = group(words, base, k4, carry)
            return carry

        zero = jnp.zeros((16,), jnp.int32)
        carry = (neg1, zero, zero)
        carry = lax.fori_loop(0, ROW_W // 16, step, carry)
        bbits, bcol, cnt = carry
        stage[0, r] = bbits
        stage[1, r] = bcol
        stage[2, r] = cnt
    pltpu.sync_copy(stage, out_hbm.at[:, pl.ds(wid * KROW, KROW)])


@functools.partial(
    pl.kernel,
    out_type=jax.ShapeDtypeStruct((3, SC_ROWS, 16), jnp.int32),
    mesh=plsc.VectorSubcoreMesh(core_axis_name="c", subcore_axis_name="s"),
    scratch_types=[
        pltpu.VMEM((ROW_W,), jnp.int32),
        pltpu.VMEM((ROW_W,), jnp.int32),
        pltpu.VMEM((3, KROW, 16), jnp.int32),
        pltpu.SemaphoreType.DMA,
        pltpu.SemaphoreType.DMA,
    ],
)
def _sc_sample(mask_hbm, out_hbm, buf0, buf1, stage, sem0, sem1):
    _sc_body(mask_hbm, out_hbm, buf0, buf1, stage, sem0, sem1)


def _sc_finish_body(raw_ref, act_ref, lp_ref):
    bits = raw_ref[0]
    col = raw_ref[1]
    cnt = raw_ref[2]
    mx = jnp.max(bits, axis=1, keepdims=True)
    act_ref[...] = jnp.min(
        jnp.where(bits == mx, col, jnp.int32(2**30)), axis=1, keepdims=True)
    total = jnp.sum(cnt, axis=1, keepdims=True)
    lp_ref[...] = -jnp.log(total.astype(jnp.float32))


@jax.jit
def _sample(mask):
    mask32 = mask.view(jnp.uint8).view(jnp.int32)
    mask_flat = jnp.pad(mask32, ((0, 0), (0, PAD_W))).reshape(-1)
    sc_raw = _sc_sample(mask_flat)
    act_tc, lp_tc = pl.pallas_call(
        _body,
        grid=(TC_ROWS // ROWS, CBLOCKS),
        in_specs=[pl.BlockSpec((ROWS, COLT), lambda r, c: (r, c))],
        out_specs=[pl.BlockSpec((ROWS, 1), lambda r, c: (r, 0)),
                   pl.BlockSpec((ROWS, 1), lambda r, c: (r, 0))],
        out_shape=[jax.ShapeDtypeStruct((TC_ROWS, 1), jnp.int32),
                   jax.ShapeDtypeStruct((TC_ROWS, 1), jnp.float32)],
        scratch_shapes=[pltpu.VMEM((ROWS, COLT), jnp.int32),
                        pltpu.VMEM((ROWS, COLT), jnp.int32)],
        compiler_params=pltpu.CompilerParams(
            dimension_semantics=("arbitrary", "arbitrary")),
    )(mask)
    act_sc, lp_sc = pl.pallas_call(
        _sc_finish_body,
        out_shape=[jax.ShapeDtypeStruct((SC_ROWS, 1), jnp.int32),
                   jax.ShapeDtypeStruct((SC_ROWS, 1), jnp.float32)],
    )(sc_raw)
    act = jnp.concatenate([act_tc[:, 0], act_sc[:, 0]])
    lp = jnp.concatenate([lp_tc[:, 0], lp_sc[:, 0]])
    return act, lp


def kernel(action_mask, fc_w, fc_b):
    del fc_w, fc_b
    return _sample(action_mask.astype(jnp.bool_))


# speedup vs baseline: 1.9474x; 1.9474x over previous
import functools

import jax
import jax.numpy as jnp
from jax import lax
from jax.experimental import pallas as pl
from jax.experimental.pallas import tpu as pltpu

BATCH = 1024
N_ACT = 100000

ROWS = 128
COLT = 4096
RSUB = 8
CSUB = 2048
CBLOCKS = (N_ACT + COLT - 1) // COLT
SB = (CBLOCKS - 1).bit_length()
KEYMASK = ((2**23 - 1) << SB) & 0x7FFFFFFF

_KS0 = 0
_KS1 = 42
_KS2 = 42 ^ 0x1BD11BDA
_ROT_A = (13, 15, 26, 6)
_ROT_B = (17, 29, 16, 24)
_INJ = ((_KS1, _KS2 + 1), (_KS2, _KS0 + 2), (_KS0, _KS1 + 3),
        (_KS1, _KS2 + 4), (_KS2, _KS0 + 5))


def _rotl(x, d):
    return lax.shift_left(x, jnp.int32(d)) | lax.shift_right_logical(
        x, jnp.int32(32 - d))


def _threefry_bits(x1):
    x0 = x1
    x1 = _rotl(x1, _ROT_A[0]) ^ x0
    first = True
    for g in range(5):
        rots = _ROT_A if g % 2 == 0 else _ROT_B
        for r in rots:
            if first:
                first = False
                continue
            x0 = x0 + x1
            x1 = _rotl(x1, r) ^ x0
        a, b = _INJ[g]
        x0 = x0 + jnp.int32(a)
        x1 = x1 + jnp.int32(b)
    return x0 ^ x1


def _body(mask_ref, act_ref, lp_ref, key_acc, cnt_acc):
    r = pl.program_id(0)
    c = pl.program_id(1)

    @pl.when(c == 0)
    def _init():
        key_acc[...] = jnp.full((ROWS, COLT), -1, jnp.int32)
        cnt_acc[...] = jnp.zeros((ROWS, COLT), jnp.int32)

    lane = lax.broadcasted_iota(jnp.int32, (RSUB, CSUB), 1)
    iota0 = lax.broadcasted_iota(jnp.int32, (RSUB, CSUB), 0)
    revstrip = jnp.int32(CBLOCKS - 1) - c
    base2d = iota0 * jnp.int32(N_ACT) + lane + jnp.int32(_KS1)
    scal0 = r * jnp.int32(ROWS * N_ACT) + c * jnp.int32(COLT)
    nchunk = (ROWS // RSUB) * (COLT // CSUB)

    def make_chunk(guarded):
        def chunk(k, _):
            ri32 = pl.multiple_of((k // (COLT // CSUB)) * 32, 32)
            ci = pl.multiple_of((k % (COLT // CSUB)) * CSUB, 256)
            mload = mask_ref[pl.ds(ri32, 32), pl.ds(ci, CSUB)] != 0
            for sub in range(4):
                ri = ri32 + sub * RSUB
                m = lax.slice(mload, (sub * RSUB, 0),
                              (sub * RSUB + RSUB, CSUB))
                if guarded:
                    valid = m & (lane < (jnp.int32(N_ACT)
                                         - c * jnp.int32(COLT) - ci))
                else:
                    valid = m
                bits = _threefry_bits(base2d + (scal0
                                                + ri * jnp.int32(N_ACT)
                                                + ci))
                key = (lax.shift_right_logical(bits, jnp.int32(9 - SB))
                       & jnp.int32(KEYMASK)) | revstrip
                v = jnp.where(valid, key, jnp.int32(-1))
                ka = key_acc[pl.ds(ri, RSUB), pl.ds(ci, CSUB)]
                key_acc[pl.ds(ri, RSUB), pl.ds(ci, CSUB)] = jnp.maximum(ka, v)
                ca = cnt_acc[pl.ds(ri, RSUB), pl.ds(ci, CSUB)]
                cnt_acc[pl.ds(ri, RSUB), pl.ds(ci, CSUB)] = \
                    ca + valid.astype(jnp.int32)
            return 0
        return chunk

    nchunk32 = (ROWS // 32) * (COLT // CSUB)

    @pl.when(c < CBLOCKS - 1)
    def _main():
        lax.fori_loop(0, nchunk32, make_chunk(False), 0)

    @pl.when(c == CBLOCKS - 1)
    def _tail():
        lax.fori_loop(0, nchunk32, make_chunk(True), 0)

    @pl.when(c == CBLOCKS - 1)
    def _fin():
        lane_f = lax.broadcasted_iota(jnp.int32, (RSUB, COLT), 1)
        for ri in range(ROWS // RSUB):
            keys = key_acc[pl.ds(ri * RSUB, RSUB), :]
            bb = lax.shift_right_arithmetic(keys, jnp.int32(SB))
            strip = jnp.int32(CBLOCKS - 1) - (keys & jnp.int32(2**SB - 1))
            gcol = strip * jnp.int32(COLT) + lane_f
            mx = jnp.max(bb, axis=1, keepdims=True)
            act_ref[pl.ds(ri * RSUB, RSUB), :] = jnp.min(
                jnp.where(bb == mx, gcol, jnp.int32(2**30)),
                axis=1, keepdims=True)
            cnt = jnp.sum(cnt_acc[pl.ds(ri * RSUB, RSUB), :],
                          axis=1, keepdims=True)
            lp_ref[pl.ds(ri * RSUB, RSUB), :] = -jnp.log(
                cnt.astype(jnp.float32))


@jax.jit
def _sample(mask):
    act, lp = pl.pallas_call(
        _body,
        grid=(BATCH // ROWS, CBLOCKS),
        in_specs=[pl.BlockSpec((ROWS, COLT), lambda r, c: (r, c))],
        out_specs=[pl.BlockSpec((ROWS, 1), lambda r, c: (r, 0)),
                   pl.BlockSpec((ROWS, 1), lambda r, c: (r, 0))],
        out_shape=[jax.ShapeDtypeStruct((BATCH, 1), jnp.int32),
                   jax.ShapeDtypeStruct((BATCH, 1), jnp.float32)],
        scratch_shapes=[pltpu.VMEM((ROWS, COLT), jnp.int32),
                        pltpu.VMEM((ROWS, COLT), jnp.int32)],
        compiler_params=pltpu.CompilerParams(
            dimension_semantics=("arbitrary", "arbitrary")),
    )(mask)
    return act[:, 0], lp[:, 0]


def kernel(action_mask, fc_w, fc_b):
    del fc_w, fc_b
    return _sample(action_mask.astype(jnp.bool_).view(jnp.uint8))
